# Initial kernel scaffold; baseline (speedup 1.0000x reference)
import jax
import jax.numpy as jnp
from jax.experimental import pallas as pl


def kernel(positions, hidden_states, ln1_w, ln2_w, Wqkv, Wo, gate_w, w1, w2, w3):
    raise NotImplementedError("write your pallas kernel here")


# speedup vs baseline: 1.6445x; 1.6445x over previous
import functools

import jax
import jax.numpy as jnp
import numpy as np
from jax import lax
from jax.experimental import pallas as pl
from jax.experimental.pallas import tpu as pltpu
from jax.experimental.pallas import tpu_sc as plsc

S = 2048
D = 1024
H = 16
KV = 8
HD = 64
FF = 3584
E = 8
TOPK = 2
EPS = 1e-5
THETA = 10000.0

BS = 512
BM = 512
BN = 512
NF = FF // BN
NT = (S * TOPK) // BM
NI = NT + E - 1

NW = 32
TPB = S // NW


def _preattn_body(pos_cos_ref, pos_sin_ref, hs_ref, ln1_ref, w_ref,
                  q_ref, k_ref, v_ref):
    hs = hs_ref[...]
    var = jnp.mean(hs * hs, axis=1, keepdims=True)
    hsn = hs * lax.rsqrt(var + EPS) * ln1_ref[...]
    qkv = jnp.dot(hsn, w_ref[...], preferred_element_type=jnp.float32)
    cos = pos_cos_ref[...]
    sin = pos_sin_ref[...]

    def rope(x):
        x1 = x[:, : HD // 2]
        x2 = x[:, HD // 2:]
        return jnp.concatenate([x1 * cos - x2 * sin, x2 * cos + x1 * sin],
                               axis=1)

    for h in range(H):
        q_ref[h] = rope(qkv[:, h * HD:(h + 1) * HD])
    for h in range(KV):
        k_ref[h] = rope(qkv[:, H * HD + h * HD: H * HD + (h + 1) * HD])
        v_ref[h] = qkv[:, (H + KV) * HD + h * HD: (H + KV) * HD + (h + 1) * HD]


def _preattn(cos_t, sin_t, hidden_states, ln1_w, Wqkv):
    bsp = 256
    nblk = S // bsp
    return pl.pallas_call(
        _preattn_body,
        grid=(nblk,),
        in_specs=[
            pl.BlockSpec((bsp, HD // 2), lambda i: (i, 0)),
            pl.BlockSpec((bsp, HD // 2), lambda i: (i, 0)),
            pl.BlockSpec((bsp, D), lambda i: (i, 0)),
            pl.BlockSpec((1, D), lambda i: (0, 0)),
            pl.BlockSpec((D, (H + 2 * KV) * HD), lambda i: (0, 0)),
        ],
        out_specs=[
            pl.BlockSpec((H, bsp, HD), lambda i: (0, i, 0)),
            pl.BlockSpec((KV, bsp, HD), lambda i: (0, i, 0)),
            pl.BlockSpec((KV, bsp, HD), lambda i: (0, i, 0)),
        ],
        out_shape=[
            jax.ShapeDtypeStruct((H, S, HD), jnp.float32),
            jax.ShapeDtypeStruct((KV, S, HD), jnp.float32),
            jax.ShapeDtypeStruct((KV, S, HD), jnp.float32),
        ],
    )(cos_t, sin_t, hidden_states, ln1_w.reshape(1, D), Wqkv)


def _attn_body(q_ref, k_ref, v_ref, o_ref):
    i = pl.program_id(1)
    q = q_ref[0]
    k = k_ref[0]
    v = v_ref[0]
    s = lax.dot_general(q, k, (((1,), (1,)), ((), ())),
                        preferred_element_type=jnp.float32) * (1.0 / 8.0)
    row = lax.broadcasted_iota(jnp.int32, (BS, S), 0) + i * BS
    col = lax.broadcasted_iota(jnp.int32, (BS, S), 1)
    s = jnp.where(col <= row, s, -1e30)
    m = jnp.max(s, axis=1, keepdims=True)
    p = jnp.exp(s - m)
    p = p / jnp.sum(p, axis=1, keepdims=True)
    o_ref[0] = jnp.dot(p, v, preferred_element_type=jnp.float32)


def _attention(q, k, v):
    rep = H // KV
    return pl.pallas_call(
        _attn_body,
        grid=(H, S // BS),
        in_specs=[
            pl.BlockSpec((1, BS, HD), lambda h, i: (h, i, 0)),
            pl.BlockSpec((1, S, HD), lambda h, i: (h // rep, 0, 0)),
            pl.BlockSpec((1, S, HD), lambda h, i: (h // rep, 0, 0)),
        ],
        out_specs=pl.BlockSpec((1, BS, HD), lambda h, i: (h, i, 0)),
        out_shape=jax.ShapeDtypeStruct((H, S, HD), jnp.float32),
    )(q, k, v)


def _postattn_body(a_ref, wo_ref, res_ref, ln2_ref, gw_ref,
                   res2_ref, hs2_ref, lg_ref):
    attn = jnp.concatenate([a_ref[h] for h in range(H)], axis=1)
    ao = jnp.dot(attn, wo_ref[...], preferred_element_type=jnp.float32)
    r2 = res_ref[...] + ao
    res2_ref[...] = r2
    var = jnp.mean(r2 * r2, axis=1, keepdims=True)
    hs2 = r2 * lax.rsqrt(var + EPS) * ln2_ref[...]
    hs2_ref[...] = hs2
    lg_ref[...] = jnp.dot(hs2, gw_ref[...], preferred_element_type=jnp.float32)


def _postattn(attn_heads, Wo, residual, ln2_w, gate_w):
    nblk = S // BS
    return pl.pallas_call(
        _postattn_body,
        grid=(nblk,),
        in_specs=[
            pl.BlockSpec((H, BS, HD), lambda i: (0, i, 0)),
            pl.BlockSpec((H * HD, D), lambda i: (0, 0)),
            pl.BlockSpec((BS, D), lambda i: (i, 0)),
            pl.BlockSpec((1, D), lambda i: (0, 0)),
            pl.BlockSpec((D, E), lambda i: (0, 0)),
        ],
        out_specs=[
            pl.BlockSpec((BS, D), lambda i: (i, 0)),
            pl.BlockSpec((BS, D), lambda i: (i, 0)),
            pl.BlockSpec((BS, E), lambda i: (i, 0)),
        ],
        out_shape=[
            jax.ShapeDtypeStruct((S, D), jnp.float32),
            jax.ShapeDtypeStruct((S, D), jnp.float32),
            jax.ShapeDtypeStruct((S, E), jnp.float32),
        ],
    )(attn_heads, Wo, residual, ln2_w.reshape(1, D), gate_w)


_RB = 512
_NP = S * TOPK


def _top2(lg):
    eid = lax.broadcasted_iota(jnp.int32, lg.shape, 1)
    m1 = jnp.max(lg, axis=1, keepdims=True)
    e1 = jnp.min(jnp.where(lg == m1, eid, E), axis=1, keepdims=True)
    lg2 = jnp.where(eid == e1, -jnp.inf, lg)
    m2 = jnp.max(lg2, axis=1, keepdims=True)
    e2 = jnp.min(jnp.where(lg2 == m2, eid, E), axis=1, keepdims=True)
    return eid, e1, e2, m1, m2


def _route_body(lgf_ref, lgb_ref, pos_ref, w_ref, cnt_ref):
    i = pl.program_id(0)
    slot = i // (S // _RB)
    eidf, e1f, e2f, _, _ = _top2(lgf_ref[...])
    ohf = jnp.concatenate(
        [(eidf == e1f), (eidf == e2f)], axis=0).astype(jnp.float32)
    eidb, e1b, e2b, m1b, m2b = _top2(lgb_ref[...])
    ohb = jnp.where(slot == 0, (eidb == e1b).astype(jnp.float32),
                    (eidb == e2b).astype(jnp.float32))
    wtop = 1.0 / (1.0 + jnp.exp(m2b - m1b))
    w_ref[...] = jnp.where(slot == 0, wtop, 1.0 - wtop)
    pairid = lax.broadcasted_iota(jnp.int32, (_RB, _NP), 0) + i * _RB
    colid = lax.broadcasted_iota(jnp.int32, (_RB, _NP), 1)
    m = (colid <= pairid).astype(jnp.float32)
    c = jnp.dot(m, ohf, preferred_element_type=jnp.float32)
    counts = jnp.sum(ohf, axis=0, keepdims=True)
    off = jnp.zeros((1, E), jnp.float32)
    for sh in range(1, E):
        off = off + jnp.concatenate(
            [jnp.zeros((1, sh), jnp.float32), counts[:, : E - sh]], axis=1)
    rank = jnp.sum(ohb * c, axis=1, keepdims=True) - 1.0
    base = jnp.sum(ohb * off, axis=1, keepdims=True)
    pos_ref[...] = (rank + base).astype(jnp.int32)
    cnt_ref[...] = counts.astype(jnp.int32)


def _route(logits):
    nblk = S // _RB
    pos, w, cnt = pl.pallas_call(
        _route_body,
        grid=(_NP // _RB,),
        in_specs=[
            pl.BlockSpec((S, E), lambda i: (0, 0)),
            pl.BlockSpec((_RB, E), lambda i: (i % (S // _RB), 0)),
        ],
        out_specs=[
            pl.BlockSpec((_RB, 1), lambda i: (i, 0)),
            pl.BlockSpec((_RB, 1), lambda i: (i, 0)),
            pl.BlockSpec((1, E), lambda i: (0, 0)),
        ],
        out_shape=[
            jax.ShapeDtypeStruct((_NP, 1), jnp.int32),
            jax.ShapeDtypeStruct((_NP, 1), jnp.float32),
            jax.ShapeDtypeStruct((1, E), jnp.int32),
        ],
    )(logits, logits)
    del nblk
    return w[:S], w[S:], pos[:S], pos[S:], cnt


@functools.cache
def _make_sc_scatter():
    mesh = plsc.VectorSubcoreMesh(core_axis_name="c", subcore_axis_name="s")

    @functools.partial(
        pl.kernel,
        out_type=jax.ShapeDtypeStruct((S * TOPK, D), jnp.float32),
        mesh=mesh,
        scratch_types=[
            pltpu.VMEM((TPB, D), jnp.float32),
            pltpu.VMEM((TPB,), jnp.int32),
            pltpu.VMEM((TPB,), jnp.int32),
            pltpu.SemaphoreType.DMA,
        ],
    )
    def sc_scatter(hs2, p0, p1, xs, rows_v, i0_v, i1_v, sem):
        wid = lax.axis_index("s") * 2 + lax.axis_index("c")
        base = wid * TPB
        pltpu.sync_copy(hs2.at[pl.ds(base, TPB)], rows_v)
        pltpu.sync_copy(p0.at[pl.ds(base, TPB)], i0_v)
        pltpu.sync_copy(p1.at[pl.ds(base, TPB)], i1_v)
        pltpu.async_copy(rows_v, xs.at[i0_v], sem).wait()
        pltpu.async_copy(rows_v, xs.at[i1_v], sem).wait()

    return sc_scatter


def _sc_scatter(hs2, p0, p1):
    return _make_sc_scatter()(hs2, p0, p1)


@functools.cache
def _make_sc_gather():
    mesh = plsc.VectorSubcoreMesh(core_axis_name="c", subcore_axis_name="s")

    @functools.partial(
        pl.kernel,
        out_type=(
            jax.ShapeDtypeStruct((S, D), jnp.float32),
            jax.ShapeDtypeStruct((S, D), jnp.float32),
        ),
        mesh=mesh,
        scratch_types=[
            pltpu.VMEM((TPB, D), jnp.float32),
            pltpu.VMEM((TPB,), jnp.int32),
            pltpu.SemaphoreType.DMA,
        ],
    )
    def sc_gather(osort, p0, p1, a0, a1, rows_v, idx_v, sem):
        wid = lax.axis_index("s") * 2 + lax.axis_index("c")
        base = wid * TPB
        pltpu.sync_copy(p0.at[pl.ds(base, TPB)], idx_v)
        pltpu.async_copy(osort.at[idx_v], rows_v, sem).wait()
        pltpu.sync_copy(rows_v, a0.at[pl.ds(base, TPB)])
        pltpu.sync_copy(p1.at[pl.ds(base, TPB)], idx_v)
        pltpu.async_copy(osort.at[idx_v], rows_v, sem).wait()
        pltpu.sync_copy(rows_v, a1.at[pl.ds(base, TPB)])

    return sc_gather


def _sc_gather(osort, p0, p1):
    return _make_sc_gather()(osort, p0, p1)


def _gmm_body(blk_ref, e_ref, gs_ref, ge_ref, rv_ref,
              x_ref, w1_ref, w3_ref, w2_ref, o_ref):
    t = pl.program_id(0)
    f = pl.program_id(1)
    rowid = lax.broadcasted_iota(jnp.int32, (BM, 1), 0) + blk_ref[t] * BM
    msk = (rowid >= gs_ref[t]) & (rowid < ge_ref[t])
    x = jnp.where(msk, x_ref[...], 0.0)
    g = jnp.dot(x, w1_ref[0], preferred_element_type=jnp.float32)
    u = jnp.dot(x, w3_ref[0], preferred_element_type=jnp.float32)
    h = g * u / (1.0 + jnp.exp(-g))
    partial = jnp.dot(h, w2_ref[0], preferred_element_type=jnp.float32)
    first = (f == 0) & (rv_ref[t] == 0)

    @pl.when(first)
    def _():
        o_ref[...] = partial

    @pl.when(jnp.logical_not(first))
    def _():
        o_ref[...] += partial


def _gmm(blk_it, e_it, gs_it, ge_it, rv_it, xs, w1, w3, w2):
    grid_spec = pltpu.PrefetchScalarGridSpec(
        num_scalar_prefetch=5,
        grid=(NI, NF),
        in_specs=[
            pl.BlockSpec((BM, D), lambda t, f, blk, e, gs, ge, rv: (blk[t], 0)),
            pl.BlockSpec((1, D, BN), lambda t, f, blk, e, gs, ge, rv: (e[t], 0, f)),
            pl.BlockSpec((1, D, BN), lambda t, f, blk, e, gs, ge, rv: (e[t], 0, f)),
            pl.BlockSpec((1, BN, D), lambda t, f, blk, e, gs, ge, rv: (e[t], f, 0)),
        ],
        out_specs=pl.BlockSpec((BM, D), lambda t, f, blk, e, gs, ge, rv: (blk[t], 0)),
    )
    return pl.pallas_call(
        _gmm_body,
        grid_spec=grid_spec,
        out_shape=jax.ShapeDtypeStruct((S * TOPK, D), jnp.float32),
    )(blk_it, e_it, gs_it, ge_it, rv_it, xs, w1, w3, w2)


def _combine_body(res2_ref, w0_ref, w1_ref, a0_ref, a1_ref, o_ref):
    o_ref[...] = (res2_ref[...] + w0_ref[...] * a0_ref[...]
                  + w1_ref[...] * a1_ref[...])


def _combine(res2, w0, w1, a0, a1):
    nblk = S // BS
    return pl.pallas_call(
        _combine_body,
        grid=(nblk,),
        in_specs=[
            pl.BlockSpec((BS, D), lambda i: (i, 0)),
            pl.BlockSpec((BS, 1), lambda i: (i, 0)),
            pl.BlockSpec((BS, 1), lambda i: (i, 0)),
            pl.BlockSpec((BS, D), lambda i: (i, 0)),
            pl.BlockSpec((BS, D), lambda i: (i, 0)),
        ],
        out_specs=pl.BlockSpec((BS, D), lambda i: (i, 0)),
        out_shape=jax.ShapeDtypeStruct((S, D), jnp.float32),
    )(res2, w0, w1, a0, a1)


def _group_metadata(counts):
    counts = counts.astype(jnp.int32)
    ge = jnp.cumsum(counts)
    gs = ge - counts
    nonzero = counts > 0
    first_blk = jnp.where(nonzero, gs // BM, 0)
    last_blk = jnp.where(nonzero, (ge - 1) // BM, -1)
    nblk = jnp.where(nonzero, last_blk - first_blk + 1, 0)
    istart = jnp.cumsum(nblk) - nblk
    total = jnp.sum(nblk)
    sidx = jnp.arange(NI, dtype=jnp.int32)
    hit = (sidx[:, None] >= istart[None, :]) & (
        sidx[:, None] < (istart + nblk)[None, :])
    e_of = jnp.sum(hit * jnp.arange(E, dtype=jnp.int32)[None, :], axis=1)
    valid = sidx < total
    blk_it = jnp.where(valid, first_blk[e_of] + sidx - istart[e_of], NT - 1)
    e_it = jnp.where(valid, e_of, E - 1)
    gs_it = jnp.where(valid, gs[e_of], 0)
    ge_it = jnp.where(valid, ge[e_of], 0)
    rv_it = jnp.concatenate([
        jnp.zeros((1,), jnp.int32),
        (blk_it[1:] == blk_it[:-1]).astype(jnp.int32),
    ])
    return blk_it, e_it, gs_it, ge_it, rv_it


def kernel(positions, hidden_states, ln1_w, ln2_w, Wqkv, Wo, gate_w, w1, w2, w3):
    inv_freq = 1.0 / (THETA ** (np.arange(0, HD, 2, dtype=np.float32) / HD))
    ang = positions.astype(jnp.float32)[:, None] * inv_freq[None, :]
    cos_t = jnp.cos(ang)
    sin_t = jnp.sin(ang)

    q, k, v = _preattn(cos_t, sin_t, hidden_states, ln1_w, Wqkv)
    attn_heads = _attention(q, k, v)
    res2, hs2, logits = _postattn(attn_heads, Wo, hidden_states, ln2_w, gate_w)
    w0, w1r, p0, p1, cnt = _route(logits)

    p0f = p0.reshape(S)
    p1f = p1.reshape(S)
    xs = _sc_scatter(hs2, p0f, p1f)

    blk_it, e_it, gs_it, ge_it, rv_it = _group_metadata(cnt[0])
    osort = _gmm(blk_it, e_it, gs_it, ge_it, rv_it, xs, w1, w3, w2)

    a0, a1 = _sc_gather(osort, p0f, p1f)
    return _combine(res2, w0, w1r, a0, a1)


# speedup vs baseline: 1.6746x; 1.0183x over previous
import functools

import jax
import jax.numpy as jnp
import numpy as np
from jax import lax
from jax.experimental import pallas as pl
from jax.experimental.pallas import tpu as pltpu
from jax.experimental.pallas import tpu_sc as plsc

S = 2048
D = 1024
H = 16
KV = 8
HD = 64
FF = 3584
E = 8
TOPK = 2
EPS = 1e-5
THETA = 10000.0

BS = 512
BM = 512
BN = 512
NF = FF // BN
NT = (S * TOPK) // BM

NW = 32
TPB = S // NW


def _preattn_body(pos_cos_ref, pos_sin_ref, hs_ref, ln1_ref, w_ref,
                  q_ref, k_ref, v_ref):
    hs = hs_ref[...]
    var = jnp.mean(hs * hs, axis=1, keepdims=True)
    hsn = hs * lax.rsqrt(var + EPS) * ln1_ref[...]
    qkv = jnp.dot(hsn, w_ref[...], preferred_element_type=jnp.float32)
    cos = pos_cos_ref[...]
    sin = pos_sin_ref[...]

    def rope(x):
        x1 = x[:, : HD // 2]
        x2 = x[:, HD // 2:]
        return jnp.concatenate([x1 * cos - x2 * sin, x2 * cos + x1 * sin],
                               axis=1)

    for h in range(H):
        q_ref[h] = rope(qkv[:, h * HD:(h + 1) * HD])
    for h in range(KV):
        k_ref[h] = rope(qkv[:, H * HD + h * HD: H * HD + (h + 1) * HD])
        v_ref[h] = qkv[:, (H + KV) * HD + h * HD: (H + KV) * HD + (h + 1) * HD]


def _preattn(cos_t, sin_t, hidden_states, ln1_w, Wqkv):
    bsp = 256
    nblk = S // bsp
    return pl.pallas_call(
        _preattn_body,
        grid=(nblk,),
        in_specs=[
            pl.BlockSpec((bsp, HD // 2), lambda i: (i, 0)),
            pl.BlockSpec((bsp, HD // 2), lambda i: (i, 0)),
            pl.BlockSpec((bsp, D), lambda i: (i, 0)),
            pl.BlockSpec((1, D), lambda i: (0, 0)),
            pl.BlockSpec((D, (H + 2 * KV) * HD), lambda i: (0, 0)),
        ],
        out_specs=[
            pl.BlockSpec((H, bsp, HD), lambda i: (0, i, 0)),
            pl.BlockSpec((KV, bsp, HD), lambda i: (0, i, 0)),
            pl.BlockSpec((KV, bsp, HD), lambda i: (0, i, 0)),
        ],
        out_shape=[
            jax.ShapeDtypeStruct((H, S, HD), jnp.float32),
            jax.ShapeDtypeStruct((KV, S, HD), jnp.float32),
            jax.ShapeDtypeStruct((KV, S, HD), jnp.float32),
        ],
    )(cos_t, sin_t, hidden_states, ln1_w.reshape(1, D), Wqkv)


def _attn_body(q_ref, k_ref, v_ref, o_ref):
    i = pl.program_id(1)
    q = q_ref[0]
    k = k_ref[0]
    v = v_ref[0]
    s = lax.dot_general(q, k, (((1,), (1,)), ((), ())),
                        preferred_element_type=jnp.float32) * (1.0 / 8.0)
    row = lax.broadcasted_iota(jnp.int32, (BS, S), 0) + i * BS
    col = lax.broadcasted_iota(jnp.int32, (BS, S), 1)
    s = jnp.where(col <= row, s, -1e30)
    m = jnp.max(s, axis=1, keepdims=True)
    p = jnp.exp(s - m)
    p = p / jnp.sum(p, axis=1, keepdims=True)
    o_ref[0] = jnp.dot(p, v, preferred_element_type=jnp.float32)


def _attention(q, k, v):
    rep = H // KV
    return pl.pallas_call(
        _attn_body,
        grid=(H, S // BS),
        in_specs=[
            pl.BlockSpec((1, BS, HD), lambda h, i: (h, i, 0)),
            pl.BlockSpec((1, S, HD), lambda h, i: (h // rep, 0, 0)),
            pl.BlockSpec((1, S, HD), lambda h, i: (h // rep, 0, 0)),
        ],
        out_specs=pl.BlockSpec((1, BS, HD), lambda h, i: (h, i, 0)),
        out_shape=jax.ShapeDtypeStruct((H, S, HD), jnp.float32),
    )(q, k, v)


def _postattn_body(a_ref, wo_ref, res_ref, ln2_ref, gw_ref,
                   res2_ref, hs2_ref, lg_ref):
    attn = jnp.concatenate([a_ref[h] for h in range(H)], axis=1)
    ao = jnp.dot(attn, wo_ref[...], preferred_element_type=jnp.float32)
    r2 = res_ref[...] + ao
    res2_ref[...] = r2
    var = jnp.mean(r2 * r2, axis=1, keepdims=True)
    hs2 = r2 * lax.rsqrt(var + EPS) * ln2_ref[...]
    hs2_ref[...] = hs2
    lg_ref[...] = jnp.dot(hs2, gw_ref[...], preferred_element_type=jnp.float32)


def _postattn(attn_heads, Wo, residual, ln2_w, gate_w):
    nblk = S // BS
    return pl.pallas_call(
        _postattn_body,
        grid=(nblk,),
        in_specs=[
            pl.BlockSpec((H, BS, HD), lambda i: (0, i, 0)),
            pl.BlockSpec((H * HD, D), lambda i: (0, 0)),
            pl.BlockSpec((BS, D), lambda i: (i, 0)),
            pl.BlockSpec((1, D), lambda i: (0, 0)),
            pl.BlockSpec((D, E), lambda i: (0, 0)),
        ],
        out_specs=[
            pl.BlockSpec((BS, D), lambda i: (i, 0)),
            pl.BlockSpec((BS, D), lambda i: (i, 0)),
            pl.BlockSpec((BS, E), lambda i: (i, 0)),
        ],
        out_shape=[
            jax.ShapeDtypeStruct((S, D), jnp.float32),
            jax.ShapeDtypeStruct((S, D), jnp.float32),
            jax.ShapeDtypeStruct((S, E), jnp.float32),
        ],
    )(attn_heads, Wo, residual, ln2_w.reshape(1, D), gate_w)


_RB = 512
_NP = S * TOPK


def _top2(lg):
    eid = lax.broadcasted_iota(jnp.int32, lg.shape, 1)
    m1 = jnp.max(lg, axis=1, keepdims=True)
    e1 = jnp.min(jnp.where(lg == m1, eid, E), axis=1, keepdims=True)
    lg2 = jnp.where(eid == e1, -jnp.inf, lg)
    m2 = jnp.max(lg2, axis=1, keepdims=True)
    e2 = jnp.min(jnp.where(lg2 == m2, eid, E), axis=1, keepdims=True)
    return eid, e1, e2, m1, m2


def _route_body(lgf_ref, lgb_ref, pos_ref, w_ref, cnt_ref):
    i = pl.program_id(0)
    slot = i // (S // _RB)
    eidf, e1f, e2f, _, _ = _top2(lgf_ref[...])
    ohf = jnp.concatenate(
        [(eidf == e1f), (eidf == e2f)], axis=0).astype(jnp.float32)
    eidb, e1b, e2b, m1b, m2b = _top2(lgb_ref[...])
    ohb = jnp.where(slot == 0, (eidb == e1b).astype(jnp.float32),
                    (eidb == e2b).astype(jnp.float32))
    wtop = 1.0 / (1.0 + jnp.exp(m2b - m1b))
    w_ref[...] = jnp.where(slot == 0, wtop, 1.0 - wtop)
    pairid = lax.broadcasted_iota(jnp.int32, (_RB, _NP), 0) + i * _RB
    colid = lax.broadcasted_iota(jnp.int32, (_RB, _NP), 1)
    m = (colid <= pairid).astype(jnp.float32)
    c = jnp.dot(m, ohf, preferred_element_type=jnp.float32)
    counts = jnp.sum(ohf, axis=0, keepdims=True)
    off = jnp.zeros((1, E), jnp.float32)
    for sh in range(1, E):
        off = off + jnp.concatenate(
            [jnp.zeros((1, sh), jnp.float32), counts[:, : E - sh]], axis=1)
    rank = jnp.sum(ohb * c, axis=1, keepdims=True) - 1.0
    base = jnp.sum(ohb * off, axis=1, keepdims=True)
    pos_ref[...] = (rank + base).astype(jnp.int32)
    cnt_ref[...] = counts.astype(jnp.int32)


def _route(logits):
    nblk = S // _RB
    pos, w, cnt = pl.pallas_call(
        _route_body,
        grid=(_NP // _RB,),
        in_specs=[
            pl.BlockSpec((S, E), lambda i: (0, 0)),
            pl.BlockSpec((_RB, E), lambda i: (i % (S // _RB), 0)),
        ],
        out_specs=[
            pl.BlockSpec((_RB, 1), lambda i: (i, 0)),
            pl.BlockSpec((_RB, 1), lambda i: (i, 0)),
            pl.BlockSpec((1, E), lambda i: (0, 0)),
        ],
        out_shape=[
            jax.ShapeDtypeStruct((_NP, 1), jnp.int32),
            jax.ShapeDtypeStruct((_NP, 1), jnp.float32),
            jax.ShapeDtypeStruct((1, E), jnp.int32),
        ],
    )(logits, logits)
    del nblk
    return w[:S], w[S:], pos[:S], pos[S:], cnt


@functools.cache
def _make_sc_scatter():
    mesh = plsc.VectorSubcoreMesh(core_axis_name="c", subcore_axis_name="s")

    @functools.partial(
        pl.kernel,
        out_type=jax.ShapeDtypeStruct((S * TOPK, D), jnp.float32),
        mesh=mesh,
        scratch_types=[
            pltpu.VMEM((TPB, D), jnp.float32),
            pltpu.VMEM((TPB,), jnp.int32),
            pltpu.VMEM((TPB,), jnp.int32),
            pltpu.SemaphoreType.DMA,
        ],
    )
    def sc_scatter(hs2, p0, p1, xs, rows_v, i0_v, i1_v, sem):
        wid = lax.axis_index("s") * 2 + lax.axis_index("c")
        base = wid * TPB
        pltpu.sync_copy(hs2.at[pl.ds(base, TPB)], rows_v)
        pltpu.sync_copy(p0.at[pl.ds(base, TPB)], i0_v)
        pltpu.sync_copy(p1.at[pl.ds(base, TPB)], i1_v)
        pltpu.async_copy(rows_v, xs.at[i0_v], sem).wait()
        pltpu.async_copy(rows_v, xs.at[i1_v], sem).wait()

    return sc_scatter


def _sc_scatter(hs2, p0, p1):
    return _make_sc_scatter()(hs2, p0, p1)


@functools.cache
def _make_sc_gather():
    mesh = plsc.VectorSubcoreMesh(core_axis_name="c", subcore_axis_name="s")

    @functools.partial(
        pl.kernel,
        out_type=(
            jax.ShapeDtypeStruct((S, D), jnp.float32),
            jax.ShapeDtypeStruct((S, D), jnp.float32),
        ),
        mesh=mesh,
        scratch_types=[
            pltpu.VMEM((TPB, D), jnp.float32),
            pltpu.VMEM((TPB,), jnp.int32),
            pltpu.SemaphoreType.DMA,
        ],
    )
    def sc_gather(osort, p0, p1, a0, a1, rows_v, idx_v, sem):
        wid = lax.axis_index("s") * 2 + lax.axis_index("c")
        base = wid * TPB
        pltpu.sync_copy(p0.at[pl.ds(base, TPB)], idx_v)
        pltpu.async_copy(osort.at[idx_v], rows_v, sem).wait()
        pltpu.sync_copy(rows_v, a0.at[pl.ds(base, TPB)])
        pltpu.sync_copy(p1.at[pl.ds(base, TPB)], idx_v)
        pltpu.async_copy(osort.at[idx_v], rows_v, sem).wait()
        pltpu.sync_copy(rows_v, a1.at[pl.ds(base, TPB)])

    return sc_gather


def _sc_gather(osort, p0, p1):
    return _make_sc_gather()(osort, p0, p1)


def _gmm_body(gs_ref, ge_ref, x_ref, w1_ref, w3_ref, w2_ref, o_ref):
    e = pl.program_id(0)
    f = pl.program_id(1)
    gs = gs_ref[e]
    ge = ge_ref[e]
    w1 = w1_ref[0]
    w3 = w3_ref[0]
    w2 = w2_ref[0]
    for b in range(NT):
        @pl.when((ge > b * BM) & (gs < (b + 1) * BM))
        def _(b=b):
            rowid = lax.broadcasted_iota(jnp.int32, (BM, 1), 0) + b * BM
            msk = (rowid >= gs) & (rowid < ge)
            x = jnp.where(msk, x_ref[pl.ds(b * BM, BM)], 0.0)
            g = jnp.dot(x, w1, preferred_element_type=jnp.float32)
            u = jnp.dot(x, w3, preferred_element_type=jnp.float32)
            h = g * u / (1.0 + jnp.exp(-g))
            partial = jnp.dot(h, w2, preferred_element_type=jnp.float32)
            first = (f == 0) & (gs <= b * BM)

            @pl.when(first)
            def _():
                o_ref[pl.ds(b * BM, BM)] = partial

            @pl.when(jnp.logical_not(first))
            def _():
                o_ref[pl.ds(b * BM, BM)] += partial


def _gmm(gs_e, ge_e, xs, w1, w3, w2):
    grid_spec = pltpu.PrefetchScalarGridSpec(
        num_scalar_prefetch=2,
        grid=(E, NF),
        in_specs=[
            pl.BlockSpec((S * TOPK, D), lambda e, f, gs, ge: (0, 0)),
            pl.BlockSpec((1, D, BN), lambda e, f, gs, ge: (e, 0, f)),
            pl.BlockSpec((1, D, BN), lambda e, f, gs, ge: (e, 0, f)),
            pl.BlockSpec((1, BN, D), lambda e, f, gs, ge: (e, f, 0)),
        ],
        out_specs=pl.BlockSpec((S * TOPK, D), lambda e, f, gs, ge: (0, 0)),
    )
    return pl.pallas_call(
        _gmm_body,
        grid_spec=grid_spec,
        out_shape=jax.ShapeDtypeStruct((S * TOPK, D), jnp.float32),
    )(gs_e, ge_e, xs, w1, w3, w2)


def _combine_body(res2_ref, w0_ref, w1_ref, a0_ref, a1_ref, o_ref):
    o_ref[...] = (res2_ref[...] + w0_ref[...] * a0_ref[...]
                  + w1_ref[...] * a1_ref[...])


def _combine(res2, w0, w1, a0, a1):
    nblk = S // BS
    return pl.pallas_call(
        _combine_body,
        grid=(nblk,),
        in_specs=[
            pl.BlockSpec((BS, D), lambda i: (i, 0)),
            pl.BlockSpec((BS, 1), lambda i: (i, 0)),
            pl.BlockSpec((BS, 1), lambda i: (i, 0)),
            pl.BlockSpec((BS, D), lambda i: (i, 0)),
            pl.BlockSpec((BS, D), lambda i: (i, 0)),
        ],
        out_specs=pl.BlockSpec((BS, D), lambda i: (i, 0)),
        out_shape=jax.ShapeDtypeStruct((S, D), jnp.float32),
    )(res2, w0, w1, a0, a1)


def _group_bounds(counts):
    counts = counts.astype(jnp.int32)
    ge = jnp.cumsum(counts)
    gs = ge - counts
    return gs, ge


def kernel(positions, hidden_states, ln1_w, ln2_w, Wqkv, Wo, gate_w, w1, w2, w3):
    inv_freq = 1.0 / (THETA ** (np.arange(0, HD, 2, dtype=np.float32) / HD))
    ang = positions.astype(jnp.float32)[:, None] * inv_freq[None, :]
    cos_t = jnp.cos(ang)
    sin_t = jnp.sin(ang)

    q, k, v = _preattn(cos_t, sin_t, hidden_states, ln1_w, Wqkv)
    attn_heads = _attention(q, k, v)
    res2, hs2, logits = _postattn(attn_heads, Wo, hidden_states, ln2_w, gate_w)
    w0, w1r, p0, p1, cnt = _route(logits)

    p0f = p0.reshape(S)
    p1f = p1.reshape(S)
    xs = _sc_scatter(hs2, p0f, p1f)

    gs_e, ge_e = _group_bounds(cnt[0])
    osort = _gmm(gs_e, ge_e, xs, w1, w3, w2)

    a0, a1 = _sc_gather(osort, p0f, p1f)
    return _combine(res2, w0, w1r, a0, a1)
